# baseline (speedup 1.0000x reference)
import jax
import jax.numpy as jnp
from jax.experimental import pallas as pl
from jax.experimental.pallas import tpu as pltpu

_BB = 8


def _emb_kernel(row_ref, col_ref, pose_ref, p_out_ref, m_out_ref, m_buf, sem):
    d = col_ref.shape[1]
    bb, _, hw = m_buf.shape
    h = 32
    w = hw // h
    B = m_out_ref.shape[0]

    ct = col_ref[1 : w + 1, :].T
    rt = row_ref[1 : h + 1, :].T
    top = jnp.broadcast_to(ct[:, None, :], (d, h, w)).reshape(d, hw)
    bot = jnp.broadcast_to(rt[:, :, None], (d, h, w)).reshape(d, hw)
    m = jnp.concatenate([top, bot], axis=0)
    m_buf[...] = jnp.broadcast_to(m[None], (bb, 2 * d, hw))

    pe = pose_ref[0, :]
    p2 = jnp.concatenate([pe, pe])
    p_out_ref[...] = jnp.broadcast_to(p2[None, :], (B, 2 * d))

    copies = [
        pltpu.make_async_copy(m_buf, m_out_ref.at[pl.ds(i * bb, bb)], sem)
        for i in range(B // bb)
    ]
    for c in copies:
        c.start()
    for c in copies:
        c.wait()


def kernel(x, row_embed, col_embed, pose_token_embed):
    B = x.shape[0]
    h, w = x.shape[-2], x.shape[-1]
    d = col_embed.shape[1]

    p_emb, m_flat = pl.pallas_call(
        _emb_kernel,
        in_specs=[
            pl.BlockSpec(memory_space=pltpu.VMEM),
            pl.BlockSpec(memory_space=pltpu.VMEM),
            pl.BlockSpec(memory_space=pltpu.VMEM),
        ],
        out_specs=[
            pl.BlockSpec(memory_space=pltpu.VMEM),
            pl.BlockSpec(memory_space=pl.ANY),
        ],
        out_shape=[
            jax.ShapeDtypeStruct((B, 2 * d), jnp.float32),
            jax.ShapeDtypeStruct((B, 2 * d, h * w), jnp.float32),
        ],
        scratch_shapes=[
            pltpu.VMEM((_BB, 2 * d, h * w), jnp.float32),
            pltpu.SemaphoreType.DMA,
        ],
    )(row_embed, col_embed, pose_token_embed)
    return (p_emb, m_flat.reshape(B, 2 * d, h, w))
